# baseline (speedup 1.0000x reference)
import jax
import jax.numpy as jnp
from jax.experimental import pallas as pl
from jax.experimental.pallas import tpu as pltpu

_FEAT = 20000
_BATCH = 32
_H1 = 512
_H2 = 256
_K = 200
_SIGN = -2**31
_R = 16
_C = _FEAT // _R
_FBLK = 2048
_NBLK = (_FEAT + _FBLK - 1) // _FBLK


def _skey(x):
    b = jax.lax.bitcast_convert_type(x, jnp.int32)
    return jnp.where(b >= 0, b, b ^ jnp.int32(0x7FFFFFFF))


def _mask_kernel(mask2d_ref, mask1d_ref, row_ref, mvec_ref):
    s2 = _skey(mask2d_ref[...])

    def body(i, p):
        bit = jnp.left_shift(jnp.int32(1), jnp.int32(31) - i)
        test = p | bit
        cnt = jnp.sum((s2 >= (test ^ jnp.int32(_SIGN))).astype(jnp.int32))
        return jnp.where(cnt >= _K, test, p)

    p = jax.lax.fori_loop(0, 32, body, jnp.int32(0), unroll=True)
    s_thr = p ^ jnp.int32(_SIGN)

    x1 = mask1d_ref[...]
    row = jnp.where(_skey(x1) >= s_thr, jax.nn.sigmoid(x1), 0.0)
    row_ref[...] = row

    x2 = mask2d_ref[...]
    row2 = jnp.where(s2 >= s_thr, jax.nn.sigmoid(x2), 0.0)
    mvec_ref[...] = jnp.broadcast_to(row2[None], (_BATCH, _R, _C))


def _mlp_kernel(feat_ref, row_ref, w1_ref, b1_ref, w2_ref, b2_ref, w3_ref,
                b3_ref, out_ref, acc_ref):
    k = pl.program_id(0)

    @pl.when(k == 0)
    def _():
        acc_ref[...] = jnp.zeros_like(acc_ref)

    dn = (((1,), (1,)), ((), ()))

    @pl.when(k < _NBLK - 1)
    def _():
        mf = feat_ref[...] * row_ref[...]
        acc_ref[...] += jax.lax.dot_general(
            mf, w1_ref[...], dn, preferred_element_type=jnp.float32)

    @pl.when(k == _NBLK - 1)
    def _():
        lim = _FEAT - (_NBLK - 1) * _FBLK
        lane = jax.lax.broadcasted_iota(jnp.int32, (1, _FBLK), 1)
        valid = lane < lim
        mf = jnp.where(valid, feat_ref[...] * row_ref[...], 0.0)
        w1 = jnp.where(valid, w1_ref[...], 0.0)
        acc = acc_ref[...] + jax.lax.dot_general(
            mf, w1, dn, preferred_element_type=jnp.float32)
        h1 = jnp.maximum(acc + b1_ref[...], 0.0)
        h2 = jnp.maximum(
            jax.lax.dot_general(h1, w2_ref[...], dn,
                                preferred_element_type=jnp.float32)
            + b2_ref[...], 0.0)
        out_ref[...] = (jnp.sum(h2 * w3_ref[...], axis=1, keepdims=True)
                        + b3_ref[0, 0])


def kernel(feature, additional, mask, W1, b1, W2, b2, W3, b3):
    del additional
    mask2d = mask.reshape(_R, _C)
    mask1d = mask.reshape(1, _FEAT)

    row1d, mvec3 = pl.pallas_call(
        _mask_kernel,
        out_shape=[
            jax.ShapeDtypeStruct((1, _FEAT), jnp.float32),
            jax.ShapeDtypeStruct((_BATCH, _R, _C), jnp.float32),
        ],
    )(mask2d, mask1d)
    mask_vector = mvec3.reshape(_BATCH, _FEAT)

    result = pl.pallas_call(
        _mlp_kernel,
        grid=(_NBLK,),
        in_specs=[
            pl.BlockSpec((_BATCH, _FBLK), lambda k: (0, k)),
            pl.BlockSpec((1, _FBLK), lambda k: (0, k)),
            pl.BlockSpec((_H1, _FBLK), lambda k: (0, k)),
            pl.BlockSpec((1, _H1), lambda k: (0, 0)),
            pl.BlockSpec((_H2, _H1), lambda k: (0, 0)),
            pl.BlockSpec((1, _H2), lambda k: (0, 0)),
            pl.BlockSpec((1, _H2), lambda k: (0, 0)),
            pl.BlockSpec((1, 1), lambda k: (0, 0)),
        ],
        out_specs=pl.BlockSpec((_BATCH, 1), lambda k: (0, 0)),
        out_shape=jax.ShapeDtypeStruct((_BATCH, 1), jnp.float32),
        scratch_shapes=[pltpu.VMEM((_BATCH, _H1), jnp.float32)],
        compiler_params=pltpu.CompilerParams(
            dimension_semantics=("arbitrary",)),
    )(feature, row1d, W1, b1.reshape(1, _H1), W2, b2.reshape(1, _H2),
      W3, b3.reshape(1, 1))

    return (result, mask_vector)


# speedup vs baseline: 1.0195x; 1.0195x over previous
import jax
import jax.numpy as jnp
from jax.experimental import pallas as pl
from jax.experimental.pallas import tpu as pltpu

_FEAT = 20000
_BATCH = 32
_H1 = 512
_H2 = 256
_K = 200
_SIGN = -2**31
_R = 16
_C = _FEAT // _R
_FBLK = 4096
_NBLK = (_FEAT + _FBLK - 1) // _FBLK


def _skey(x):
    b = jax.lax.bitcast_convert_type(x, jnp.int32)
    return jnp.where(b >= 0, b, b ^ jnp.int32(0x7FFFFFFF))


def _mask_kernel(mask2d_ref, mask1d_ref, row_ref, mvec_ref):
    s2 = _skey(mask2d_ref[...])

    def body(i, p):
        bit = jnp.left_shift(jnp.int32(1), jnp.int32(31) - i)
        test = p | bit
        cnt = jnp.sum((s2 >= (test ^ jnp.int32(_SIGN))).astype(jnp.int32))
        return jnp.where(cnt >= _K, test, p)

    p = jax.lax.fori_loop(0, 32, body, jnp.int32(0), unroll=True)
    s_thr = p ^ jnp.int32(_SIGN)

    x1 = mask1d_ref[...]
    row = jnp.where(_skey(x1) >= s_thr, jax.nn.sigmoid(x1), 0.0)
    row_ref[...] = row

    x2 = mask2d_ref[...]
    row2 = jnp.where(s2 >= s_thr, jax.nn.sigmoid(x2), 0.0)
    mvec_ref[...] = jnp.broadcast_to(row2[None], (_BATCH, _R, _C))


def _mlp_kernel(feat_ref, row_ref, w1_ref, b1_ref, w2_ref, b2_ref, w3_ref,
                b3_ref, out_ref, acc_ref):
    k = pl.program_id(0)

    @pl.when(k == 0)
    def _():
        acc_ref[...] = jnp.zeros_like(acc_ref)

    dn = (((1,), (1,)), ((), ()))

    @pl.when(k < _NBLK - 1)
    def _():
        mf = feat_ref[...] * row_ref[...]
        acc_ref[...] += jax.lax.dot_general(
            mf, w1_ref[...], dn, preferred_element_type=jnp.float32)

    @pl.when(k == _NBLK - 1)
    def _():
        lim = _FEAT - (_NBLK - 1) * _FBLK
        lane = jax.lax.broadcasted_iota(jnp.int32, (1, _FBLK), 1)
        valid = lane < lim
        mf = jnp.where(valid, feat_ref[...] * row_ref[...], 0.0)
        w1 = jnp.where(valid, w1_ref[...], 0.0)
        acc = acc_ref[...] + jax.lax.dot_general(
            mf, w1, dn, preferred_element_type=jnp.float32)
        h1 = jnp.maximum(acc + b1_ref[...], 0.0)
        h2 = jnp.maximum(
            jax.lax.dot_general(h1, w2_ref[...], dn,
                                preferred_element_type=jnp.float32)
            + b2_ref[...], 0.0)
        out_ref[...] = (jnp.sum(h2 * w3_ref[...], axis=1, keepdims=True)
                        + b3_ref[0, 0])


def kernel(feature, additional, mask, W1, b1, W2, b2, W3, b3):
    del additional
    mask2d = mask.reshape(_R, _C)
    mask1d = mask.reshape(1, _FEAT)

    row1d, mvec3 = pl.pallas_call(
        _mask_kernel,
        out_shape=[
            jax.ShapeDtypeStruct((1, _FEAT), jnp.float32),
            jax.ShapeDtypeStruct((_BATCH, _R, _C), jnp.float32),
        ],
    )(mask2d, mask1d)
    mask_vector = mvec3.reshape(_BATCH, _FEAT)

    result = pl.pallas_call(
        _mlp_kernel,
        grid=(_NBLK,),
        in_specs=[
            pl.BlockSpec((_BATCH, _FBLK), lambda k: (0, k)),
            pl.BlockSpec((1, _FBLK), lambda k: (0, k)),
            pl.BlockSpec((_H1, _FBLK), lambda k: (0, k)),
            pl.BlockSpec((1, _H1), lambda k: (0, 0)),
            pl.BlockSpec((_H2, _H1), lambda k: (0, 0)),
            pl.BlockSpec((1, _H2), lambda k: (0, 0)),
            pl.BlockSpec((1, _H2), lambda k: (0, 0)),
            pl.BlockSpec((1, 1), lambda k: (0, 0)),
        ],
        out_specs=pl.BlockSpec((_BATCH, 1), lambda k: (0, 0)),
        out_shape=jax.ShapeDtypeStruct((_BATCH, 1), jnp.float32),
        scratch_shapes=[pltpu.VMEM((_BATCH, _H1), jnp.float32)],
        compiler_params=pltpu.CompilerParams(
            dimension_semantics=("arbitrary",)),
    )(feature, row1d, W1, b1.reshape(1, _H1), W2, b2.reshape(1, _H2),
      W3, b3.reshape(1, 1))

    return (result, mask_vector)
